# Initial kernel scaffold; baseline (speedup 1.0000x reference)
import jax
import jax.numpy as jnp
from jax.experimental import pallas as pl


def kernel(cat_indices, numeric, tables, norm_mean, norm_std, W1, b1, W2, b2):
    raise NotImplementedError("write your pallas kernel here")


# speedup vs baseline: 7.8619x; 7.8619x over previous
import functools

import jax
import jax.numpy as jnp
from jax import lax
from jax.experimental import pallas as pl
from jax.experimental.pallas import tpu as pltpu
from jax.experimental.pallas import tpu_sc as plsc

B = 16384
F_CAT = 26
F_NUM = 13
V = 100000
D = 16
H = 32

_NC = 2
_NS = 16
_NW = _NC * _NS
_R = B * F_CAT
_RPW = _R // _NW
_CHUNK = 128
_K = 13
_GROUP = _K * _CHUNK
_NGROUP = _RPW // _GROUP
assert _NGROUP * _GROUP == _RPW


def _gather_body(tab_hbm, idx_hbm, out_hbm, idx_v, rows_v, sem):
    wid = lax.axis_index("s") * _NC + lax.axis_index("c")
    chunk_base = wid * (_RPW // _CHUNK)
    row_base = wid * _RPW
    pltpu.sync_copy(idx_hbm.at[pl.ds(chunk_base, _RPW // _CHUNK)], idx_v)

    descs = [None] * _NGROUP

    def fire(g, buf):
        ds = []
        for j in range(_K):
            c = g * _K + j
            d = pltpu.async_copy(
                tab_hbm.at[idx_v.at[c]],
                rows_v.at[buf].at[pl.ds(j * _CHUNK, _CHUNK)],
                sem,
            )
            ds.append(d)
        return ds

    def drain_and_store(g, buf):
        for d in descs[g]:
            d.wait()
        pltpu.sync_copy(
            rows_v.at[buf],
            out_hbm.at[pl.ds(row_base + g * _GROUP, _GROUP)],
        )

    descs[0] = fire(0, 0)
    for g in range(1, _NGROUP):
        descs[g] = fire(g, g % 2)
        drain_and_store(g - 1, (g - 1) % 2)
    drain_and_store(_NGROUP - 1, (_NGROUP - 1) % 2)


def _sc_gather(tab_flat, idx_chunks):
    mesh = plsc.VectorSubcoreMesh(
        core_axis_name="c", subcore_axis_name="s",
        num_cores=_NC, num_subcores=_NS,
    )
    fn = pl.kernel(
        _gather_body,
        out_type=jax.ShapeDtypeStruct((_R, D), jnp.float32),
        mesh=mesh,
        scratch_types=[
            pltpu.VMEM((_RPW // _CHUNK, _CHUNK), jnp.int32),
            pltpu.VMEM((2, _GROUP, D), jnp.float32),
            pltpu.SemaphoreType.DMA,
        ],
        compiler_params=pltpu.CompilerParams(use_tc_tiling_on_sc=False),
    )
    return fn(tab_flat, idx_chunks)


def _mlp_body(emb, num, mean, std, w1e, w1n, b1, w2, b2, out):
    nn = (num[...] - mean[...]) / std[...]
    x = jnp.dot(emb[...], w1e[...], preferred_element_type=jnp.float32)
    x = x + jnp.dot(nn, w1n[...], preferred_element_type=jnp.float32)
    x = jnp.maximum(x + b1[...], 0.0)
    out[...] = jnp.dot(x, w2[...], preferred_element_type=jnp.float32) + b2[...]


def _mlp(emb_flat, numeric, norm_mean, norm_std, W1, b1, W2, b2):
    BT = 2048
    E = F_CAT * D
    return pl.pallas_call(
        _mlp_body,
        grid=(B // BT,),
        in_specs=[
            pl.BlockSpec((BT, E), lambda i: (i, 0)),
            pl.BlockSpec((BT, F_NUM), lambda i: (i, 0)),
            pl.BlockSpec((1, F_NUM), lambda i: (0, 0)),
            pl.BlockSpec((1, F_NUM), lambda i: (0, 0)),
            pl.BlockSpec((E, H), lambda i: (0, 0)),
            pl.BlockSpec((F_NUM, H), lambda i: (0, 0)),
            pl.BlockSpec((1, H), lambda i: (0, 0)),
            pl.BlockSpec((H, 1), lambda i: (0, 0)),
            pl.BlockSpec((1, 1), lambda i: (0, 0)),
        ],
        out_specs=pl.BlockSpec((BT, 1), lambda i: (i, 0)),
        out_shape=jax.ShapeDtypeStruct((B, 1), jnp.float32),
    )(
        emb_flat, numeric,
        norm_mean.reshape(1, F_NUM), norm_std.reshape(1, F_NUM),
        W1[: F_CAT * D], W1[F_CAT * D:],
        b1.reshape(1, H), W2, b2.reshape(1, 1),
    )


def kernel(cat_indices, numeric, tables, norm_mean, norm_std, W1, b1, W2, b2):
    offs = (jnp.arange(F_CAT, dtype=jnp.int32) * V)[None, :]
    idx = (cat_indices.astype(jnp.int32) + offs).reshape(_R // _CHUNK, _CHUNK)
    tab_flat = tables.reshape(F_CAT * V, D)
    rows = _sc_gather(tab_flat, idx)
    emb_flat = rows.reshape(B, F_CAT * D)
    return _mlp(emb_flat, numeric, norm_mean, norm_std, W1, b1, W2, b2)


# speedup vs baseline: 36.4536x; 4.6368x over previous
import jax
import jax.numpy as jnp
from jax import lax
from jax.experimental import pallas as pl
from jax.experimental.pallas import tpu as pltpu
from jax.experimental.pallas import tpu_sc as plsc

B = 16384
F_CAT = 26
F_NUM = 13
V = 100000
D = 16
H = 32

_NC = 2
_NS = 16
_NW = _NC * _NS
_NR = F_CAT * D
_RPW = _NR // _NW
_HALF = B // 2


def _gather_body(tab_hbm, idx_hbm, out_hbm, row_v, idx_v, out_v):
    wid = lax.axis_index("s") * _NC + lax.axis_index("c")
    for j in range(_RPW):
        r = wid * _RPW + j
        f = r // D
        pltpu.sync_copy(idx_hbm.at[f], idx_v)
        pltpu.sync_copy(tab_hbm.at[r], row_v)
        for h in range(2):
            def body(k, _):
                base = h * _HALF + k * 16
                vals = plsc.load_gather(row_v, [idx_v[pl.ds(base, 16)]])
                out_v[pl.ds(k * 16, 16)] = vals
                return _
            lax.fori_loop(0, _HALF // 16, body, 0, unroll=8)
            pltpu.sync_copy(out_v, out_hbm.at[r, pl.ds(h * _HALF, _HALF)])


def _sc_gather(tab_rows, idx_cols):
    mesh = plsc.VectorSubcoreMesh(
        core_axis_name="c", subcore_axis_name="s",
        num_cores=_NC, num_subcores=_NS,
    )
    fn = pl.kernel(
        _gather_body,
        out_type=jax.ShapeDtypeStruct((_NR, B), jnp.float32),
        mesh=mesh,
        scratch_types=[
            pltpu.VMEM((V,), jnp.float32),
            pltpu.VMEM((B,), jnp.int32),
            pltpu.VMEM((_HALF,), jnp.float32),
        ],
        compiler_params=pltpu.CompilerParams(
            use_tc_tiling_on_sc=True, needs_layout_passes=False,
        ),
    )
    return fn(tab_rows, idx_cols)


def _mlp_body(embT, numT, meanc, stdc, w1eT, w1nT, b1c, w2T, b2c, outT):
    nn = (numT[...] - meanc[...]) / stdc[...]
    x = jnp.dot(w1eT[...], embT[...], preferred_element_type=jnp.float32)
    x = x + jnp.dot(w1nT[...], nn, preferred_element_type=jnp.float32)
    x = jnp.maximum(x + b1c[...], 0.0)
    outT[...] = jnp.dot(w2T[...], x, preferred_element_type=jnp.float32) + b2c[...]


def _mlp(embT, numeric, norm_mean, norm_std, W1, b1, W2, b2):
    BT = 2048
    E = F_CAT * D
    outT = pl.pallas_call(
        _mlp_body,
        grid=(B // BT,),
        in_specs=[
            pl.BlockSpec((E, BT), lambda i: (0, i)),
            pl.BlockSpec((F_NUM, BT), lambda i: (0, i)),
            pl.BlockSpec((F_NUM, 1), lambda i: (0, 0)),
            pl.BlockSpec((F_NUM, 1), lambda i: (0, 0)),
            pl.BlockSpec((H, E), lambda i: (0, 0)),
            pl.BlockSpec((H, F_NUM), lambda i: (0, 0)),
            pl.BlockSpec((H, 1), lambda i: (0, 0)),
            pl.BlockSpec((1, H), lambda i: (0, 0)),
            pl.BlockSpec((1, 1), lambda i: (0, 0)),
        ],
        out_specs=pl.BlockSpec((1, BT), lambda i: (0, i)),
        out_shape=jax.ShapeDtypeStruct((1, B), jnp.float32),
    )(
        embT, jnp.transpose(numeric),
        norm_mean.reshape(F_NUM, 1), norm_std.reshape(F_NUM, 1),
        jnp.transpose(W1[:E]), jnp.transpose(W1[E:]),
        b1.reshape(H, 1), jnp.transpose(W2), b2.reshape(1, 1),
    )
    return outT.reshape(B, 1)


def kernel(cat_indices, numeric, tables, norm_mean, norm_std, W1, b1, W2, b2):
    tab_rows = jnp.transpose(tables, (0, 2, 1)).reshape(_NR, V)
    idx_cols = jnp.transpose(cat_indices).astype(jnp.int32)
    embT = _sc_gather(tab_rows, idx_cols)
    return _mlp(embT, numeric, norm_mean, norm_std, W1, b1, W2, b2)


# speedup vs baseline: 61.0798x; 1.6756x over previous
import jax
import jax.numpy as jnp
from jax import lax
from jax.experimental import pallas as pl
from jax.experimental.pallas import tpu as pltpu
from jax.experimental.pallas import tpu_sc as plsc

B = 16384
F_CAT = 26
F_NUM = 13
V = 100000
D = 16
H = 32

_NC = 2
_NS = 16
_NW = _NC * _NS
_NR = F_CAT * D
_RPW = _NR // _NW
_HALF = B // 2


def _gather_body(tab_hbm, idx_hbm, out_hbm, row_v, idx_v, out_v):
    wid = lax.axis_index("s") * _NC + lax.axis_index("c")
    for j in range(_RPW):
        r = wid * _RPW + j
        f = r // D
        pltpu.sync_copy(idx_hbm.at[f], idx_v)
        pltpu.sync_copy(tab_hbm.at[r], row_v)
        for h in range(2):
            @plsc.parallel_loop(0, _HALF // 16, unroll=8)
            def _gather_iter(k, h=h):
                base = h * _HALF + k * 16
                vals = plsc.load_gather(row_v, [idx_v[pl.ds(base, 16)]])
                out_v[pl.ds(k * 16, 16)] = vals
            pltpu.sync_copy(out_v, out_hbm.at[r, pl.ds(h * _HALF, _HALF)])


def _sc_gather(tab_rows, idx_cols):
    mesh = plsc.VectorSubcoreMesh(
        core_axis_name="c", subcore_axis_name="s",
        num_cores=_NC, num_subcores=_NS,
    )
    fn = pl.kernel(
        _gather_body,
        out_type=jax.ShapeDtypeStruct((_NR, B), jnp.float32),
        mesh=mesh,
        scratch_types=[
            pltpu.VMEM((V,), jnp.float32),
            pltpu.VMEM((B,), jnp.int32),
            pltpu.VMEM((_HALF,), jnp.float32),
        ],
        compiler_params=pltpu.CompilerParams(
            use_tc_tiling_on_sc=True, needs_layout_passes=False,
        ),
    )
    return fn(tab_rows, idx_cols)


def _mlp_body(embT, numT, meanc, stdc, w1eT, w1nT, b1c, w2T, b2c, outT):
    nn = (numT[...] - meanc[...]) / stdc[...]
    x = jnp.dot(w1eT[...], embT[...], preferred_element_type=jnp.float32)
    x = x + jnp.dot(w1nT[...], nn, preferred_element_type=jnp.float32)
    x = jnp.maximum(x + b1c[...], 0.0)
    outT[...] = jnp.dot(w2T[...], x, preferred_element_type=jnp.float32) + b2c[...]


def _mlp(embT, numeric, norm_mean, norm_std, W1, b1, W2, b2):
    BT = 2048
    E = F_CAT * D
    outT = pl.pallas_call(
        _mlp_body,
        grid=(B // BT,),
        in_specs=[
            pl.BlockSpec((E, BT), lambda i: (0, i)),
            pl.BlockSpec((F_NUM, BT), lambda i: (0, i)),
            pl.BlockSpec((F_NUM, 1), lambda i: (0, 0)),
            pl.BlockSpec((F_NUM, 1), lambda i: (0, 0)),
            pl.BlockSpec((H, E), lambda i: (0, 0)),
            pl.BlockSpec((H, F_NUM), lambda i: (0, 0)),
            pl.BlockSpec((H, 1), lambda i: (0, 0)),
            pl.BlockSpec((1, H), lambda i: (0, 0)),
            pl.BlockSpec((1, 1), lambda i: (0, 0)),
        ],
        out_specs=pl.BlockSpec((1, BT), lambda i: (0, i)),
        out_shape=jax.ShapeDtypeStruct((1, B), jnp.float32),
    )(
        embT, jnp.transpose(numeric),
        norm_mean.reshape(F_NUM, 1), norm_std.reshape(F_NUM, 1),
        jnp.transpose(W1[:E]), jnp.transpose(W1[E:]),
        b1.reshape(H, 1), jnp.transpose(W2), b2.reshape(1, 1),
    )
    return outT.reshape(B, 1)


def kernel(cat_indices, numeric, tables, norm_mean, norm_std, W1, b1, W2, b2):
    tab_rows = jnp.transpose(tables, (0, 2, 1)).reshape(_NR, V)
    idx_cols = jnp.transpose(cat_indices).astype(jnp.int32)
    embT = _sc_gather(tab_rows, idx_cols)
    return _mlp(embT, numeric, norm_mean, norm_std, W1, b1, W2, b2)
